# baseline (speedup 1.0000x reference)
import functools

import jax
import jax.numpy as jnp
from jax import lax
from jax.experimental import pallas as pl
from jax.experimental.pallas import tpu as pltpu
from jax.experimental.pallas import tpu_sc as plsc

_LOG2 = 0.6931471805599453
_NC = 2
_NS = 16
_C = 40
_IB = 10
_NPAD = 10240


def _ssp(v):
    return jnp.maximum(v, 0.0) + jnp.log1p(jnp.exp(-jnp.abs(v))) - _LOG2


def _h_body(x_ref, w_ref, o_ref):
    o_ref[...] = jnp.dot(x_ref[...], w_ref[...],
                         preferred_element_type=jnp.float32)


def _wij_body(f_ref, rc_ref, wf1_ref, bf1_ref, wf2_ref, bf2_ref, o_ref):
    t = jnp.dot(f_ref[...], wf1_ref[...],
                preferred_element_type=jnp.float32) + bf1_ref[...]
    t = _ssp(t)
    t = jnp.dot(t.astype(jnp.bfloat16), wf2_ref[...],
                preferred_element_type=jnp.float32) + bf2_ref[...]
    eb, d = t.shape
    t3 = t.reshape(eb // 8, 8, d)
    rcb = lax.broadcast_in_dim(rc_ref[...], (eb // 8, 8, d), (0, 1))
    o_ref[...] = (t3 * rcb).reshape(eb, d)


def _out_body(n, p_ref, wo1_ref, bo1_ref, wo2_ref, bo2_ref, o_ref):
    agg = p_ref[pl.ds(0, n), :] + p_ref[pl.ds(_NPAD, n), :]
    t = _ssp(jnp.dot(agg.astype(jnp.bfloat16), wo1_ref[...],
                     preferred_element_type=jnp.float32) + bo1_ref[...])
    o_ref[...] = jnp.dot(t.astype(jnp.bfloat16), wo2_ref[...],
                         preferred_element_type=jnp.float32) + bo2_ref[...]


def _sc_body(e, d, h_hbm, wij_hbm, idxi_hbm, idxj_hbm, out_hbm,
             idxi_v, idxj_v, rows_v, wijb_v, xij_v, agg_sh,
             gsems, wsems, ssems):
    c = lax.axis_index("c")
    s = lax.axis_index("s")
    wid = c * _NS + s

    ept = e // (_NC * _NS)
    nch = ept // _C
    rows_per_tile = _NPAD // _NS
    nslc = d // 16

    zvec = jnp.zeros((16,), jnp.float32)

    def _zfill(rr, carry):
        for t in range(nslc):
            xij_v[0, rr, pl.ds(t * 16, 16)] = zvec
        return carry
    lax.fori_loop(0, _C, _zfill, 0)
    for i in range(rows_per_tile // _C):
        pltpu.sync_copy(xij_v.at[0],
                        agg_sh.at[pl.ds(s * rows_per_tile + i * _C, _C)])
    plsc.subcore_barrier()

    ebase = wid * ept

    pltpu.sync_copy(idxi_hbm.at[wid, 0], idxi_v.at[0])
    pltpu.sync_copy(idxj_hbm.at[wid, 0], idxj_v.at[0])
    for sl in (0, 1):
        pltpu.async_copy(h_hbm.at[idxj_v.at[0, sl]], rows_v.at[sl],
                         gsems.at[sl])
        pltpu.async_copy(wij_hbm.at[pl.ds(ebase + sl * _C, _C)],
                         wijb_v.at[sl], wsems.at[sl])

    def _pair(i, carry):
        for sl in (0, 1):
            k = 2 * i + sl
            pltpu.make_async_copy(h_hbm.at[pl.ds(0, _C)], rows_v.at[sl],
                                  gsems.at[sl]).wait()
            pltpu.make_async_copy(wij_hbm.at[pl.ds(0, _C)], wijb_v.at[sl],
                                  wsems.at[sl]).wait()

            @pl.when(k >= 2)
            def _():
                pltpu.make_async_copy(xij_v.at[sl],
                                      agg_sh.at[pl.ds(0, _C)],
                                      ssems.at[sl]).wait()

            def _mul(ei, cc):
                for t in range(nslc):
                    slc = pl.ds(t * 16, 16)
                    xij_v[sl, ei, slc] = (rows_v[sl, ei, slc]
                                          * wijb_v[sl, ei, slc])
                return cc
            lax.fori_loop(0, _C, _mul, 0)

            kn = k + 2
            blkn = kn // _IB
            pbn = lax.rem(blkn, 2)
            krn = kn - blkn * _IB

            @pl.when(kn < nch)
            def _():
                @pl.when(krn == 0)
                def _():
                    pltpu.sync_copy(idxi_hbm.at[wid, blkn], idxi_v.at[pbn])
                    pltpu.sync_copy(idxj_hbm.at[wid, blkn], idxj_v.at[pbn])
                pltpu.async_copy(h_hbm.at[idxj_v.at[pbn, krn]],
                                 rows_v.at[sl], gsems.at[sl])
                pltpu.async_copy(wij_hbm.at[pl.ds(ebase + kn * _C, _C)],
                                 wijb_v.at[sl], wsems.at[sl])

            blk = k // _IB
            pb = lax.rem(blk, 2)
            krow = k - blk * _IB
            pltpu.async_copy(xij_v.at[sl], agg_sh.at[idxi_v.at[pb, krow]],
                             ssems.at[sl], add=True)
        return carry
    lax.fori_loop(0, nch // 2, _pair, 0)

    for sl in (0, 1):
        pltpu.make_async_copy(xij_v.at[sl], agg_sh.at[pl.ds(0, _C)],
                              ssems.at[sl]).wait()

    plsc.subcore_barrier()

    r0 = s * rows_per_tile
    pltpu.sync_copy(agg_sh.at[pl.ds(r0, rows_per_tile)],
                    out_hbm.at[pl.ds(c * _NPAD + r0, rows_per_tile)])


def kernel(x, f_ij, idx_i, idx_j, rcut_ij, W_in2f, Wf1, bf1, Wf2, bf2,
           Wo1, bo1, Wo2, bo2):
    n, d = x.shape
    e, r = f_ij.shape
    f = Wf2.shape[1]
    nw = _NC * _NS
    assert (e % (nw * _C * _IB) == 0 and (e // (nw * _C)) % 2 == 0
            and n <= _NPAD and d % 16 == 0)

    h = pl.pallas_call(
        _h_body,
        out_shape=jax.ShapeDtypeStruct((n, f), jnp.float32),
    )(x.astype(jnp.bfloat16), W_in2f.astype(jnp.bfloat16))

    eb = 32000
    wij = pl.pallas_call(
        _wij_body,
        grid=(e // eb,),
        in_specs=[
            pl.BlockSpec((eb, r), lambda i: (i, 0)),
            pl.BlockSpec((eb // 8, 8), lambda i: (i, 0)),
            pl.BlockSpec((r, f), lambda i: (0, 0)),
            pl.BlockSpec((1, f), lambda i: (0, 0)),
            pl.BlockSpec((f, f), lambda i: (0, 0)),
            pl.BlockSpec((1, f), lambda i: (0, 0)),
        ],
        out_specs=pl.BlockSpec((eb, f), lambda i: (i, 0)),
        out_shape=jax.ShapeDtypeStruct((e, f), jnp.float32),
    )(f_ij.astype(jnp.bfloat16), rcut_ij.reshape(e // 8, 8),
      Wf1.astype(jnp.bfloat16), bf1.reshape(1, f),
      Wf2.astype(jnp.bfloat16), bf2.reshape(1, f))

    mesh = plsc.VectorSubcoreMesh(core_axis_name="c", subcore_axis_name="s",
                                  num_cores=_NC, num_subcores=_NS)
    nch = e // (nw * _C)
    nblk = nch // _IB
    sc = pl.kernel(
        functools.partial(_sc_body, e, f),
        out_type=jax.ShapeDtypeStruct((_NC * _NPAD, f), jnp.float32),
        mesh=mesh,
        scratch_types=[
            pltpu.VMEM((2, _IB, _C), jnp.int32),
            pltpu.VMEM((2, _IB, _C), jnp.int32),
            pltpu.VMEM((2, _C, f), jnp.float32),
            pltpu.VMEM((2, _C, f), jnp.float32),
            pltpu.VMEM((2, _C, f), jnp.float32),
            pltpu.VMEM_SHARED((_NPAD, f), jnp.float32),
            pltpu.SemaphoreType.DMA((2,)),
            pltpu.SemaphoreType.DMA((2,)),
            pltpu.SemaphoreType.DMA((2,)),
        ],
    )
    partials = sc(h, wij,
                  idx_i.astype(jnp.int32).reshape(nw, nblk, _IB, _C),
                  idx_j.astype(jnp.int32).reshape(nw, nblk, _IB, _C))

    out = pl.pallas_call(
        functools.partial(_out_body, n),
        out_shape=jax.ShapeDtypeStruct((n, d), jnp.float32),
    )(partials, Wo1.astype(jnp.bfloat16), bo1.reshape(1, d),
      Wo2.astype(jnp.bfloat16), bo2.reshape(1, d))
    return out


# speedup vs baseline: 1.0023x; 1.0023x over previous
import functools

import jax
import jax.numpy as jnp
from jax import lax
from jax.experimental import pallas as pl
from jax.experimental.pallas import tpu as pltpu
from jax.experimental.pallas import tpu_sc as plsc

_LOG2 = 0.6931471805599453
_NC = 2
_NS = 16
_C = 40
_IB = 10
_NPAD = 10240


def _ssp(v):
    return jnp.maximum(v, 0.0) + jnp.log1p(jnp.exp(-jnp.abs(v))) - _LOG2


def _h_body(x_ref, w_ref, o_ref):
    o_ref[...] = jnp.dot(x_ref[...], w_ref[...],
                         preferred_element_type=jnp.float32)


def _wij_body(f_ref, rc_ref, wf1_ref, bf1_ref, wf2_ref, bf2_ref, o_ref):
    t = jnp.dot(f_ref[...], wf1_ref[...],
                preferred_element_type=jnp.float32) + bf1_ref[...]
    t = _ssp(t)
    t = jnp.dot(t.astype(jnp.bfloat16), wf2_ref[...],
                preferred_element_type=jnp.float32) + bf2_ref[...]
    eb, d = t.shape
    t3 = t.reshape(eb // 8, 8, d)
    rcb = lax.broadcast_in_dim(rc_ref[...], (eb // 8, 8, d), (0, 1))
    o_ref[...] = (t3 * rcb).reshape(eb, d)


def _out_body(n, p_ref, wo1_ref, bo1_ref, wo2_ref, bo2_ref, o_ref):
    agg = p_ref[pl.ds(0, n), :] + p_ref[pl.ds(_NPAD, n), :]
    t = _ssp(jnp.dot(agg.astype(jnp.bfloat16), wo1_ref[...],
                     preferred_element_type=jnp.float32) + bo1_ref[...])
    o_ref[...] = jnp.dot(t.astype(jnp.bfloat16), wo2_ref[...],
                         preferred_element_type=jnp.float32) + bo2_ref[...]


def _sc_body(e, d, h_hbm, wij_hbm, idxi_hbm, idxj_hbm, out_hbm,
             idxi_v, idxj_v, rows_v, wijb_v, xij_v, agg_sh,
             gsems, wsems, ssems):
    c = lax.axis_index("c")
    s = lax.axis_index("s")
    wid = c * _NS + s

    ept = e // (_NC * _NS)
    nch = ept // _C
    rows_per_tile = _NPAD // _NS
    nslc = d // 16

    zvec = jnp.zeros((16,), jnp.float32)

    def _zfill(rr, carry):
        for t in range(nslc):
            xij_v[0, rr, pl.ds(t * 16, 16)] = zvec
        return carry
    lax.fori_loop(0, _C, _zfill, 0)
    for i in range(rows_per_tile // _C):
        pltpu.sync_copy(xij_v.at[0],
                        agg_sh.at[pl.ds(s * rows_per_tile + i * _C, _C)])
    plsc.subcore_barrier()

    ebase = wid * ept

    pltpu.sync_copy(idxi_hbm.at[wid, 0], idxi_v.at[0])
    pltpu.sync_copy(idxj_hbm.at[wid, 0], idxj_v.at[0])
    for sl in (0, 1):
        pltpu.async_copy(h_hbm.at[idxj_v.at[0, sl]], rows_v.at[sl],
                         gsems.at[sl])
        pltpu.async_copy(wij_hbm.at[pl.ds(ebase + sl * _C, _C)],
                         wijb_v.at[sl], wsems.at[sl])

    def _pair(i, carry):
        for sl in (0, 1):
            k = 2 * i + sl
            pltpu.make_async_copy(h_hbm.at[pl.ds(0, _C)], rows_v.at[sl],
                                  gsems.at[sl]).wait()
            pltpu.make_async_copy(wij_hbm.at[pl.ds(0, _C)], wijb_v.at[sl],
                                  wsems.at[sl]).wait()

            @pl.when(k >= 2)
            def _():
                pltpu.make_async_copy(xij_v.at[sl],
                                      agg_sh.at[pl.ds(0, _C)],
                                      ssems.at[sl]).wait()

            def _mul(ei, cc):
                for t in range(nslc):
                    slc = pl.ds(t * 16, 16)
                    xij_v[sl, ei, slc] = (rows_v[sl, ei, slc]
                                          * wijb_v[sl, ei, slc])
                return cc
            lax.fori_loop(0, _C, _mul, 0)

            kn = k + 2
            blkn = kn // _IB
            pbn = lax.rem(blkn, 2)
            krn = kn - blkn * _IB

            @pl.when(kn < nch)
            def _():
                @pl.when(krn == 0)
                def _():
                    pltpu.sync_copy(idxi_hbm.at[wid, blkn], idxi_v.at[pbn])
                    pltpu.sync_copy(idxj_hbm.at[wid, blkn], idxj_v.at[pbn])
                pltpu.async_copy(h_hbm.at[idxj_v.at[pbn, krn]],
                                 rows_v.at[sl], gsems.at[sl])
                pltpu.async_copy(wij_hbm.at[pl.ds(ebase + kn * _C, _C)],
                                 wijb_v.at[sl], wsems.at[sl])

            blk = k // _IB
            pb = lax.rem(blk, 2)
            krow = k - blk * _IB
            pltpu.async_copy(xij_v.at[sl], agg_sh.at[idxi_v.at[pb, krow]],
                             ssems.at[sl], add=True)
        return carry
    lax.fori_loop(0, nch // 2, _pair, 0)

    for sl in (0, 1):
        pltpu.make_async_copy(xij_v.at[sl], agg_sh.at[pl.ds(0, _C)],
                              ssems.at[sl]).wait()

    plsc.subcore_barrier()

    r0 = s * rows_per_tile
    pltpu.sync_copy(agg_sh.at[pl.ds(r0, rows_per_tile)],
                    out_hbm.at[pl.ds(c * _NPAD + r0, rows_per_tile)])


def kernel(x, f_ij, idx_i, idx_j, rcut_ij, W_in2f, Wf1, bf1, Wf2, bf2,
           Wo1, bo1, Wo2, bo2):
    n, d = x.shape
    e, r = f_ij.shape
    f = Wf2.shape[1]
    nw = _NC * _NS
    assert (e % (nw * _C * _IB) == 0 and (e // (nw * _C)) % 2 == 0
            and n <= _NPAD and d % 16 == 0)

    h = pl.pallas_call(
        _h_body,
        out_shape=jax.ShapeDtypeStruct((n, f), jnp.float32),
    )(x.astype(jnp.bfloat16), W_in2f.astype(jnp.bfloat16))

    eb = 16000
    wij = pl.pallas_call(
        _wij_body,
        grid=(e // eb,),
        in_specs=[
            pl.BlockSpec((eb, r), lambda i: (i, 0)),
            pl.BlockSpec((eb // 8, 8), lambda i: (i, 0)),
            pl.BlockSpec((r, f), lambda i: (0, 0)),
            pl.BlockSpec((1, f), lambda i: (0, 0)),
            pl.BlockSpec((f, f), lambda i: (0, 0)),
            pl.BlockSpec((1, f), lambda i: (0, 0)),
        ],
        out_specs=pl.BlockSpec((eb, f), lambda i: (i, 0)),
        out_shape=jax.ShapeDtypeStruct((e, f), jnp.float32),
    )(f_ij.astype(jnp.bfloat16), rcut_ij.reshape(e // 8, 8),
      Wf1.astype(jnp.bfloat16), bf1.reshape(1, f),
      Wf2.astype(jnp.bfloat16), bf2.reshape(1, f))

    mesh = plsc.VectorSubcoreMesh(core_axis_name="c", subcore_axis_name="s",
                                  num_cores=_NC, num_subcores=_NS)
    nch = e // (nw * _C)
    nblk = nch // _IB
    sc = pl.kernel(
        functools.partial(_sc_body, e, f),
        out_type=jax.ShapeDtypeStruct((_NC * _NPAD, f), jnp.float32),
        mesh=mesh,
        scratch_types=[
            pltpu.VMEM((2, _IB, _C), jnp.int32),
            pltpu.VMEM((2, _IB, _C), jnp.int32),
            pltpu.VMEM((2, _C, f), jnp.float32),
            pltpu.VMEM((2, _C, f), jnp.float32),
            pltpu.VMEM((2, _C, f), jnp.float32),
            pltpu.VMEM_SHARED((_NPAD, f), jnp.float32),
            pltpu.SemaphoreType.DMA((2,)),
            pltpu.SemaphoreType.DMA((2,)),
            pltpu.SemaphoreType.DMA((2,)),
        ],
    )
    partials = sc(h, wij,
                  idx_i.astype(jnp.int32).reshape(nw, nblk, _IB, _C),
                  idx_j.astype(jnp.int32).reshape(nw, nblk, _IB, _C))

    out = pl.pallas_call(
        functools.partial(_out_body, n),
        out_shape=jax.ShapeDtypeStruct((n, d), jnp.float32),
    )(partials, Wo1.astype(jnp.bfloat16), bo1.reshape(1, d),
      Wo2.astype(jnp.bfloat16), bo2.reshape(1, d))
    return out
